# baseline (speedup 1.0000x reference)
import functools

import jax
import jax.numpy as jnp
from jax import lax
from jax.experimental import pallas as pl
from jax.experimental.pallas import tpu as pltpu
from jax.experimental.pallas import tpu_sc as plsc

NC = 2
NS = 16
NW = NC * NS

N = 10000
E = 320000
D = 128

NPAD = 10240
RPT = NPAD // NS
EPW = E // NW
CHUNK = 80
NCHUNK = EPW // CHUNK

ROWS = 1000
GRID = N // ROWS

_MESH = plsc.VectorSubcoreMesh(
    core_axis_name="c", subcore_axis_name="s", num_cores=NC, num_subcores=NS
)


@functools.partial(
    pl.kernel,
    out_type=jax.ShapeDtypeStruct((NC, NPAD), jnp.float32),
    mesh=_MESH,
    scratch_types=[
        pltpu.VMEM_SHARED((NPAD,), jnp.float32),
        pltpu.VMEM((NCHUNK, CHUNK), jnp.int32),
        pltpu.VMEM((CHUNK,), jnp.float32),
        pltpu.VMEM((RPT,), jnp.float32),
        pltpu.SemaphoreType.DMA,
    ],
)
def _deg_kernel(dst3_hbm, out_hbm, acc, didx, ones, zbuf, semz):
    c = lax.axis_index("c")
    s = lax.axis_index("s")
    wid = s * NC + c

    pltpu.async_copy(dst3_hbm.at[wid], didx, semz)
    for i in range(CHUNK // 16):
        ones[pl.ds(i * 16, 16)] = jnp.ones((16,), jnp.float32)

    def zb(i, carry):
        zbuf[pl.ds(i * 16, 16)] = jnp.zeros((16,), jnp.float32)
        return carry

    lax.fori_loop(0, RPT // 16, zb, 0)
    pltpu.sync_copy(zbuf, acc.at[pl.ds(s * RPT, RPT)])
    pltpu.make_async_copy(dst3_hbm.at[wid], didx, semz).wait()
    plsc.subcore_barrier()

    G = 5

    def fire(g):
        for t in range(G):
            pltpu.async_copy(ones, acc.at[didx.at[g * G + t]], semz, add=True)

    def drain(g):
        for t in range(G):
            pltpu.make_async_copy(ones, acc.at[didx.at[g * G + t]], semz).wait()

    fire(0)

    def body(i, carry):
        fire(i + 1)
        drain(i)
        return carry

    lax.fori_loop(0, NCHUNK // G - 1, body, 0)
    drain(NCHUNK // G - 1)
    plsc.subcore_barrier()
    pltpu.sync_copy(acc.at[pl.ds(s * RPT, RPT)], out_hbm.at[c, pl.ds(s * RPT, RPT)])


@functools.partial(
    pl.kernel,
    out_type=jax.ShapeDtypeStruct((NC, NPAD, D), jnp.float32),
    mesh=_MESH,
    scratch_types=[
        pltpu.VMEM_SHARED((NPAD, D), jnp.float32),
        pltpu.VMEM((EPW,), jnp.int32),
        pltpu.VMEM((NCHUNK, CHUNK), jnp.int32),
        pltpu.VMEM((CHUNK, D), jnp.float32),
        pltpu.VMEM((CHUNK, D), jnp.float32),
        pltpu.SemaphoreType.DMA,
        pltpu.SemaphoreType.DMA,
        pltpu.SemaphoreType.DMA,
        pltpu.SemaphoreType.DMA,
    ],
)
def _seg_kernel(xs_hbm, src2_hbm, dst3_hbm, out_hbm, acc, sidx, didx,
                rows0, rows1, sem0, sem1, sems0, sems1):
    c = lax.axis_index("c")
    s = lax.axis_index("s")
    wid = s * NC + c

    pltpu.async_copy(src2_hbm.at[wid], sidx, sem0)
    pltpu.async_copy(dst3_hbm.at[wid], didx, sem1)

    def zb(i, carry):
        for b in range(D // 16):
            rows0[i, pl.ds(b * 16, 16)] = jnp.zeros((16,), jnp.float32)
        return carry

    lax.fori_loop(0, CHUNK, zb, 0)
    for k in range(RPT // CHUNK):
        pltpu.sync_copy(rows0, acc.at[pl.ds(s * RPT + k * CHUNK, CHUNK)])
    pltpu.make_async_copy(src2_hbm.at[wid], sidx, sem0).wait()
    pltpu.make_async_copy(dst3_hbm.at[wid], didx, sem1).wait()
    plsc.subcore_barrier()

    def gather(j, rows, sem):
        pltpu.async_copy(xs_hbm.at[sidx.at[pl.ds(j * CHUNK, CHUNK)]], rows, sem)

    def gwait(j, rows, sem):
        pltpu.make_async_copy(
            xs_hbm.at[sidx.at[pl.ds(j * CHUNK, CHUNK)]], rows, sem
        ).wait()

    def scat(j, rows, sem):
        pltpu.async_copy(rows, acc.at[didx.at[j]], sem, add=True)

    def swait(j, rows, sem):
        pltpu.make_async_copy(rows, acc.at[didx.at[j]], sem).wait()

    gather(0, rows0, sem0)
    gather(1, rows1, sem1)

    def body(i, carry):
        j0 = 2 * i
        gwait(j0, rows0, sem0)
        scat(j0, rows0, sems0)
        gwait(j0 + 1, rows1, sem1)
        scat(j0 + 1, rows1, sems1)
        swait(j0, rows0, sems0)
        gather(j0 + 2, rows0, sem0)
        swait(j0 + 1, rows1, sems1)
        gather(j0 + 3, rows1, sem1)
        return carry

    lax.fori_loop(0, (NCHUNK - 3) // 2, body, 0)
    j = NCHUNK - 3
    gwait(j, rows0, sem0)
    scat(j, rows0, sems0)
    gwait(j + 1, rows1, sem1)
    scat(j + 1, rows1, sems1)
    swait(j, rows0, sems0)
    gather(j + 2, rows0, sem0)
    swait(j + 1, rows1, sems1)
    gwait(j + 2, rows0, sem0)
    scat(j + 2, rows0, sems0)
    swait(j + 2, rows0, sems0)
    plsc.subcore_barrier()
    pltpu.sync_copy(acc.at[pl.ds(s * RPT, RPT)], out_hbm.at[c, pl.ds(s * RPT, RPT)])


def _xs_body(x_ref, d0_ref, d1_ref, xs_ref, dv_ref):
    dinv = lax.rsqrt(d0_ref[...] + d1_ref[...] + 1.0)
    xs_ref[...] = x_ref[...] * dinv
    dv_ref[...] = dinv


_xs_call = pl.pallas_call(
    _xs_body,
    grid=(GRID,),
    in_specs=[
        pl.BlockSpec((ROWS, D), lambda i: (i, 0)),
        pl.BlockSpec((ROWS, 1), lambda i: (i, 0)),
        pl.BlockSpec((ROWS, 1), lambda i: (i, 0)),
    ],
    out_specs=[
        pl.BlockSpec((ROWS, D), lambda i: (i, 0)),
        pl.BlockSpec((ROWS, 1), lambda i: (i, 0)),
    ],
    out_shape=[
        jax.ShapeDtypeStruct((N, D), jnp.float32),
        jax.ShapeDtypeStruct((N, 1), jnp.float32),
    ],
)


def _out_body(s_ref, xs_ref, dv_ref, wmu_ref, bmu_ref, wls_ref, bls_ref,
              mu_ref, ls_ref):
    t = s_ref[0] + s_ref[1] + xs_ref[...]
    dinv = dv_ref[...]
    mu = jnp.dot(t, wmu_ref[...], preferred_element_type=jnp.float32)
    ls = jnp.dot(t, wls_ref[...], preferred_element_type=jnp.float32)
    mu_ref[...] = dinv * mu + bmu_ref[...]
    ls_ref[...] = dinv * ls + bls_ref[...]


_out_call = pl.pallas_call(
    _out_body,
    grid=(GRID,),
    in_specs=[
        pl.BlockSpec((NC, ROWS, D), lambda i: (0, i, 0)),
        pl.BlockSpec((ROWS, D), lambda i: (i, 0)),
        pl.BlockSpec((ROWS, 1), lambda i: (i, 0)),
        pl.BlockSpec((D, D), lambda i: (0, 0)),
        pl.BlockSpec((1, D), lambda i: (0, 0)),
        pl.BlockSpec((D, D), lambda i: (0, 0)),
        pl.BlockSpec((1, D), lambda i: (0, 0)),
    ],
    out_specs=[
        pl.BlockSpec((ROWS, D), lambda i: (i, 0)),
        pl.BlockSpec((ROWS, D), lambda i: (i, 0)),
    ],
    out_shape=[
        jax.ShapeDtypeStruct((N, D), jnp.float32),
        jax.ShapeDtypeStruct((N, D), jnp.float32),
    ],
)


def kernel(x, edge_index, W_mu, b_mu, W_logstd, b_logstd):
    src2 = edge_index[0].reshape(NW, EPW)
    dst3 = edge_index[1].reshape(NW, NCHUNK, CHUNK)
    degp = _deg_kernel(dst3)
    d0 = degp[0].reshape(NPAD, 1)
    d1 = degp[1].reshape(NPAD, 1)

    xs, dv = _xs_call(x, d0, d1)
    s = _seg_kernel(xs, src2, dst3)
    mu, ls = _out_call(s, xs, dv,
                       W_mu, b_mu.reshape(1, D), W_logstd, b_logstd.reshape(1, D))
    return mu, ls


# speedup vs baseline: 1.1909x; 1.1909x over previous
import functools

import jax
import jax.numpy as jnp
from jax import lax
from jax.experimental import pallas as pl
from jax.experimental.pallas import tpu as pltpu
from jax.experimental.pallas import tpu_sc as plsc

NC = 2
NS = 16
NW = NC * NS

N = 10000
E = 320000
D = 128

NPAD = 10240
RPT = NPAD // NS
EPW = E // NW
CHUNK = 80
NCHUNK = EPW // CHUNK

ROWS = 1000
GRID = N // ROWS

_MESH = plsc.VectorSubcoreMesh(
    core_axis_name="c", subcore_axis_name="s", num_cores=NC, num_subcores=NS
)


@functools.partial(
    pl.kernel,
    out_type=jax.ShapeDtypeStruct((NC, NPAD), jnp.float32),
    mesh=_MESH,
    scratch_types=[
        pltpu.VMEM_SHARED((NPAD,), jnp.float32),
        pltpu.VMEM((NCHUNK, CHUNK), jnp.int32),
        pltpu.VMEM((CHUNK,), jnp.float32),
        pltpu.VMEM((RPT,), jnp.float32),
        pltpu.SemaphoreType.DMA,
    ],
)
def _deg_kernel(dst3_hbm, out_hbm, acc, didx, ones, zbuf, semz):
    c = lax.axis_index("c")
    s = lax.axis_index("s")
    wid = s * NC + c

    pltpu.async_copy(dst3_hbm.at[wid], didx, semz)
    for i in range(CHUNK // 16):
        ones[pl.ds(i * 16, 16)] = jnp.ones((16,), jnp.float32)

    def zb(i, carry):
        zbuf[pl.ds(i * 16, 16)] = jnp.zeros((16,), jnp.float32)
        return carry

    lax.fori_loop(0, RPT // 16, zb, 0)
    pltpu.sync_copy(zbuf, acc.at[pl.ds(s * RPT, RPT)])
    pltpu.make_async_copy(dst3_hbm.at[wid], didx, semz).wait()
    plsc.subcore_barrier()

    G = 5

    def fire(g):
        for t in range(G):
            pltpu.async_copy(ones, acc.at[didx.at[g * G + t]], semz, add=True)

    def drain(g):
        for t in range(G):
            pltpu.make_async_copy(ones, acc.at[didx.at[g * G + t]], semz).wait()

    fire(0)

    def body(i, carry):
        fire(i + 1)
        drain(i)
        return carry

    lax.fori_loop(0, NCHUNK // G - 1, body, 0)
    drain(NCHUNK // G - 1)
    plsc.subcore_barrier()
    pltpu.sync_copy(acc.at[pl.ds(s * RPT, RPT)], out_hbm.at[c, pl.ds(s * RPT, RPT)])


@functools.partial(
    pl.kernel,
    out_type=jax.ShapeDtypeStruct((NC, NPAD, D), jnp.float32),
    mesh=_MESH,
    scratch_types=[
        pltpu.VMEM_SHARED((NPAD, D), jnp.float32),
        pltpu.VMEM((EPW,), jnp.int32),
        pltpu.VMEM((NCHUNK, CHUNK), jnp.int32),
        pltpu.VMEM((CHUNK, D), jnp.float32),
        pltpu.VMEM((CHUNK, D), jnp.float32),
        pltpu.SemaphoreType.DMA,
        pltpu.SemaphoreType.DMA,
    ],
)
def _seg_kernel(xs_hbm, src2_hbm, dst3_hbm, out_hbm, acc, sidx, didx,
                rows0, rows1, sem0, sem1):
    c = lax.axis_index("c")
    s = lax.axis_index("s")
    wid = s * NC + c

    pltpu.async_copy(src2_hbm.at[wid], sidx, sem0)
    pltpu.async_copy(dst3_hbm.at[wid], didx, sem1)

    def zb(i, carry):
        for b in range(D // 16):
            rows0[i, pl.ds(b * 16, 16)] = jnp.zeros((16,), jnp.float32)
        return carry

    lax.fori_loop(0, CHUNK, zb, 0)
    for k in range(RPT // CHUNK):
        pltpu.sync_copy(rows0, acc.at[pl.ds(s * RPT + k * CHUNK, CHUNK)])
    pltpu.make_async_copy(src2_hbm.at[wid], sidx, sem0).wait()
    pltpu.make_async_copy(dst3_hbm.at[wid], didx, sem1).wait()
    plsc.subcore_barrier()

    def gather(j, rows, sem):
        pltpu.async_copy(xs_hbm.at[sidx.at[pl.ds(j * CHUNK, CHUNK)]], rows, sem)

    def gwait(j, rows, sem):
        pltpu.make_async_copy(
            xs_hbm.at[sidx.at[pl.ds(j * CHUNK, CHUNK)]], rows, sem
        ).wait()

    gather(0, rows0, sem0)

    def body(i, carry):
        j0 = 2 * i
        gather(j0 + 1, rows1, sem1)
        gwait(j0, rows0, sem0)
        pltpu.sync_copy(rows0, acc.at[didx.at[j0]], add=True)
        gather(j0 + 2, rows0, sem0)
        gwait(j0 + 1, rows1, sem1)
        pltpu.sync_copy(rows1, acc.at[didx.at[j0 + 1]], add=True)
        return carry

    lax.fori_loop(0, (NCHUNK - 1) // 2, body, 0)
    gwait(NCHUNK - 1, rows0, sem0)
    pltpu.sync_copy(rows0, acc.at[didx.at[NCHUNK - 1]], add=True)
    plsc.subcore_barrier()
    pltpu.sync_copy(acc.at[pl.ds(s * RPT, RPT)], out_hbm.at[c, pl.ds(s * RPT, RPT)])


def _xs_body(x_ref, d0_ref, d1_ref, xs_ref, dv_ref):
    dinv = lax.rsqrt(d0_ref[...] + d1_ref[...] + 1.0)
    xs_ref[...] = x_ref[...] * dinv
    dv_ref[...] = dinv


_xs_call = pl.pallas_call(
    _xs_body,
    grid=(GRID,),
    in_specs=[
        pl.BlockSpec((ROWS, D), lambda i: (i, 0)),
        pl.BlockSpec((ROWS, 1), lambda i: (i, 0)),
        pl.BlockSpec((ROWS, 1), lambda i: (i, 0)),
    ],
    out_specs=[
        pl.BlockSpec((ROWS, D), lambda i: (i, 0)),
        pl.BlockSpec((ROWS, 1), lambda i: (i, 0)),
    ],
    out_shape=[
        jax.ShapeDtypeStruct((N, D), jnp.float32),
        jax.ShapeDtypeStruct((N, 1), jnp.float32),
    ],
)


def _out_body(s_ref, xs_ref, dv_ref, wmu_ref, bmu_ref, wls_ref, bls_ref,
              mu_ref, ls_ref):
    t = s_ref[0] + s_ref[1] + xs_ref[...]
    dinv = dv_ref[...]
    mu = jnp.dot(t, wmu_ref[...], preferred_element_type=jnp.float32)
    ls = jnp.dot(t, wls_ref[...], preferred_element_type=jnp.float32)
    mu_ref[...] = dinv * mu + bmu_ref[...]
    ls_ref[...] = dinv * ls + bls_ref[...]


_out_call = pl.pallas_call(
    _out_body,
    grid=(GRID,),
    in_specs=[
        pl.BlockSpec((NC, ROWS, D), lambda i: (0, i, 0)),
        pl.BlockSpec((ROWS, D), lambda i: (i, 0)),
        pl.BlockSpec((ROWS, 1), lambda i: (i, 0)),
        pl.BlockSpec((D, D), lambda i: (0, 0)),
        pl.BlockSpec((1, D), lambda i: (0, 0)),
        pl.BlockSpec((D, D), lambda i: (0, 0)),
        pl.BlockSpec((1, D), lambda i: (0, 0)),
    ],
    out_specs=[
        pl.BlockSpec((ROWS, D), lambda i: (i, 0)),
        pl.BlockSpec((ROWS, D), lambda i: (i, 0)),
    ],
    out_shape=[
        jax.ShapeDtypeStruct((N, D), jnp.float32),
        jax.ShapeDtypeStruct((N, D), jnp.float32),
    ],
)


def kernel(x, edge_index, W_mu, b_mu, W_logstd, b_logstd):
    src2 = edge_index[0].reshape(NW, EPW)
    dst3 = edge_index[1].reshape(NW, NCHUNK, CHUNK)
    degp = _deg_kernel(dst3)
    d0 = degp[0].reshape(NPAD, 1)
    d1 = degp[1].reshape(NPAD, 1)

    xs, dv = _xs_call(x, d0, d1)
    s = _seg_kernel(xs, src2, dst3)
    mu, ls = _out_call(s, xs, dv,
                       W_mu, b_mu.reshape(1, D), W_logstd, b_logstd.reshape(1, D))
    return mu, ls


# speedup vs baseline: 1.1920x; 1.0009x over previous
import functools

import jax
import jax.numpy as jnp
from jax import lax
from jax.experimental import pallas as pl
from jax.experimental.pallas import tpu as pltpu
from jax.experimental.pallas import tpu_sc as plsc

NC = 2
NS = 16
NW = NC * NS

N = 10000
E = 320000
D = 128

NPAD = 10240
RPT = NPAD // NS
EPW = E // NW
CHUNK = 80
NCHUNK = EPW // CHUNK

ROWS = 1000
GRID = N // ROWS

_MESH = plsc.VectorSubcoreMesh(
    core_axis_name="c", subcore_axis_name="s", num_cores=NC, num_subcores=NS
)


@functools.partial(
    pl.kernel,
    out_type=jax.ShapeDtypeStruct((NC, NPAD), jnp.float32),
    mesh=_MESH,
    scratch_types=[
        pltpu.VMEM_SHARED((NPAD,), jnp.float32),
        pltpu.VMEM((NCHUNK, CHUNK), jnp.int32),
        pltpu.VMEM((CHUNK,), jnp.float32),
        pltpu.VMEM((RPT,), jnp.float32),
        pltpu.SemaphoreType.DMA,
    ],
)
def _deg_kernel(dst3_hbm, out_hbm, acc, didx, ones, zbuf, semz):
    c = lax.axis_index("c")
    s = lax.axis_index("s")
    wid = s * NC + c

    pltpu.async_copy(dst3_hbm.at[wid], didx, semz)
    for i in range(CHUNK // 16):
        ones[pl.ds(i * 16, 16)] = jnp.ones((16,), jnp.float32)

    def zb(i, carry):
        zbuf[pl.ds(i * 16, 16)] = jnp.zeros((16,), jnp.float32)
        return carry

    lax.fori_loop(0, RPT // 16, zb, 0)
    pltpu.sync_copy(zbuf, acc.at[pl.ds(s * RPT, RPT)])
    pltpu.make_async_copy(dst3_hbm.at[wid], didx, semz).wait()
    plsc.subcore_barrier()

    G = 25

    def fire(g):
        for t in range(G):
            pltpu.async_copy(ones, acc.at[didx.at[g * G + t]], semz, add=True)

    def drain(g):
        for t in range(G):
            pltpu.make_async_copy(ones, acc.at[didx.at[g * G + t]], semz).wait()

    fire(0)

    def body(i, carry):
        fire(i + 1)
        drain(i)
        return carry

    lax.fori_loop(0, NCHUNK // G - 1, body, 0)
    drain(NCHUNK // G - 1)
    plsc.subcore_barrier()
    pltpu.sync_copy(acc.at[pl.ds(s * RPT, RPT)], out_hbm.at[c, pl.ds(s * RPT, RPT)])


@functools.partial(
    pl.kernel,
    out_type=jax.ShapeDtypeStruct((NC, NPAD, D), jnp.float32),
    mesh=_MESH,
    scratch_types=[
        pltpu.VMEM_SHARED((NPAD, D), jnp.float32),
        pltpu.VMEM((EPW,), jnp.int32),
        pltpu.VMEM((NCHUNK, CHUNK), jnp.int32),
        pltpu.VMEM((CHUNK, D), jnp.float32),
        pltpu.VMEM((CHUNK, D), jnp.float32),
        pltpu.SemaphoreType.DMA,
        pltpu.SemaphoreType.DMA,
    ],
)
def _seg_kernel(xs_hbm, src2_hbm, dst3_hbm, out_hbm, acc, sidx, didx,
                rows0, rows1, sem0, sem1):
    c = lax.axis_index("c")
    s = lax.axis_index("s")
    wid = s * NC + c

    pltpu.async_copy(src2_hbm.at[wid], sidx, sem0)
    pltpu.async_copy(dst3_hbm.at[wid], didx, sem1)

    def zb(i, carry):
        for b in range(D // 16):
            rows0[i, pl.ds(b * 16, 16)] = jnp.zeros((16,), jnp.float32)
        return carry

    lax.fori_loop(0, CHUNK, zb, 0)
    for k in range(RPT // CHUNK):
        pltpu.sync_copy(rows0, acc.at[pl.ds(s * RPT + k * CHUNK, CHUNK)])
    pltpu.make_async_copy(src2_hbm.at[wid], sidx, sem0).wait()
    pltpu.make_async_copy(dst3_hbm.at[wid], didx, sem1).wait()
    plsc.subcore_barrier()

    def gather(j, rows, sem):
        pltpu.async_copy(xs_hbm.at[sidx.at[pl.ds(j * CHUNK, CHUNK)]], rows, sem)

    def gwait(j, rows, sem):
        pltpu.make_async_copy(
            xs_hbm.at[sidx.at[pl.ds(j * CHUNK, CHUNK)]], rows, sem
        ).wait()

    gather(0, rows0, sem0)

    def body(i, carry):
        j0 = 2 * i
        gather(j0 + 1, rows1, sem1)
        gwait(j0, rows0, sem0)
        pltpu.sync_copy(rows0, acc.at[didx.at[j0]], add=True)
        gather(j0 + 2, rows0, sem0)
        gwait(j0 + 1, rows1, sem1)
        pltpu.sync_copy(rows1, acc.at[didx.at[j0 + 1]], add=True)
        return carry

    lax.fori_loop(0, (NCHUNK - 1) // 2, body, 0)
    gwait(NCHUNK - 1, rows0, sem0)
    pltpu.sync_copy(rows0, acc.at[didx.at[NCHUNK - 1]], add=True)
    plsc.subcore_barrier()
    pltpu.sync_copy(acc.at[pl.ds(s * RPT, RPT)], out_hbm.at[c, pl.ds(s * RPT, RPT)])


def _xs_body(x_ref, d0_ref, d1_ref, xs_ref, dv_ref):
    dinv = lax.rsqrt(d0_ref[...] + d1_ref[...] + 1.0)
    xs_ref[...] = x_ref[...] * dinv
    dv_ref[...] = dinv


_xs_call = pl.pallas_call(
    _xs_body,
    grid=(GRID,),
    in_specs=[
        pl.BlockSpec((ROWS, D), lambda i: (i, 0)),
        pl.BlockSpec((ROWS, 1), lambda i: (i, 0)),
        pl.BlockSpec((ROWS, 1), lambda i: (i, 0)),
    ],
    out_specs=[
        pl.BlockSpec((ROWS, D), lambda i: (i, 0)),
        pl.BlockSpec((ROWS, 1), lambda i: (i, 0)),
    ],
    out_shape=[
        jax.ShapeDtypeStruct((N, D), jnp.float32),
        jax.ShapeDtypeStruct((N, 1), jnp.float32),
    ],
)


def _out_body(s_ref, xs_ref, dv_ref, wmu_ref, bmu_ref, wls_ref, bls_ref,
              mu_ref, ls_ref):
    t = s_ref[0] + s_ref[1] + xs_ref[...]
    dinv = dv_ref[...]
    mu = jnp.dot(t, wmu_ref[...], preferred_element_type=jnp.float32)
    ls = jnp.dot(t, wls_ref[...], preferred_element_type=jnp.float32)
    mu_ref[...] = dinv * mu + bmu_ref[...]
    ls_ref[...] = dinv * ls + bls_ref[...]


_out_call = pl.pallas_call(
    _out_body,
    grid=(GRID,),
    in_specs=[
        pl.BlockSpec((NC, ROWS, D), lambda i: (0, i, 0)),
        pl.BlockSpec((ROWS, D), lambda i: (i, 0)),
        pl.BlockSpec((ROWS, 1), lambda i: (i, 0)),
        pl.BlockSpec((D, D), lambda i: (0, 0)),
        pl.BlockSpec((1, D), lambda i: (0, 0)),
        pl.BlockSpec((D, D), lambda i: (0, 0)),
        pl.BlockSpec((1, D), lambda i: (0, 0)),
    ],
    out_specs=[
        pl.BlockSpec((ROWS, D), lambda i: (i, 0)),
        pl.BlockSpec((ROWS, D), lambda i: (i, 0)),
    ],
    out_shape=[
        jax.ShapeDtypeStruct((N, D), jnp.float32),
        jax.ShapeDtypeStruct((N, D), jnp.float32),
    ],
)


def kernel(x, edge_index, W_mu, b_mu, W_logstd, b_logstd):
    src2 = edge_index[0].reshape(NW, EPW)
    dst3 = edge_index[1].reshape(NW, NCHUNK, CHUNK)
    degp = _deg_kernel(dst3)
    d0 = degp[0].reshape(NPAD, 1)
    d1 = degp[1].reshape(NPAD, 1)

    xs, dv = _xs_call(x, d0, d1)
    s = _seg_kernel(xs, src2, dst3)
    mu, ls = _out_call(s, xs, dv,
                       W_mu, b_mu.reshape(1, D), W_logstd, b_logstd.reshape(1, D))
    return mu, ls
